# baseline (speedup 1.0000x reference)
import functools

import jax
import jax.numpy as jnp
from jax.experimental import pallas as pl
from jax.experimental.pallas import tpu as pltpu

_ACTION_WEIGHT = 10.0
_ACTION_DIM = 8
_OBS_DIM = 128
_TARGET_WEIGHT = 3.0


def _entities_block(pm, bbatch, ne, od):
    ad = _ACTION_DIM
    f32 = jnp.float32
    c = pm[:, :ne * od].reshape(bbatch * ne, od)
    main = c[:, ad:]
    wrapsrc = jnp.concatenate(
        [c[1:, :ad], jnp.zeros((1, ad), f32)], axis=0)
    tails = pm[:, ne * od:ne * od + ad]
    trep = jnp.broadcast_to(tails[:, None, :],
                            (bbatch, ne, ad)).reshape(bbatch * ne, ad)
    rows = jax.lax.broadcasted_iota(jnp.int32, (bbatch * ne, 1), 0)
    islast = (rows % ne) == (ne - 1)
    wrap = jnp.where(islast, trep, wrapsrc)
    return jnp.concatenate([main, wrap], axis=1)


def _chamfer_body(horizon, bbatch, ne, od, px_ref, py_ref,
                  out_ref, acc1_ref, acc2_ref, asum_ref):
    g = pl.program_id(0)
    nsteps = pl.num_programs(0)
    f32 = jnp.float32

    iif = jax.lax.broadcasted_iota(jnp.int32, (ne, ne), 0).astype(f32)
    jjf = jax.lax.broadcasted_iota(jnp.int32, (ne, ne), 1).astype(f32)
    onesm = jnp.ones((ne, od), f32)

    @pl.when(g == 0)
    def _init():
        acc1_ref[:] = jnp.zeros_like(acc1_ref)
        acc2_ref[:] = jnp.zeros_like(acc2_ref)
        asum_ref[0] = f32(0.0)
        asum_ref[1] = f32(0.0)

    px = px_ref[:]
    py = py_ref[:]
    ex = _entities_block(px, bbatch, ne, od)
    ey = _entities_block(py, bbatch, ne, od)

    acc1 = acc1_ref[:]
    acc2 = acc2_ref[:]
    for r in range(bbatch):
        x = ex[r * ne:(r + 1) * ne]
        y = ey[r * ne:(r + 1) * ne]
        xa = jnp.concatenate([x * x, x], axis=1)
        ya = jnp.concatenate([y * y, y], axis=1)
        xb = jnp.concatenate([onesm, -2.0 * x], axis=1)
        yb = jnp.concatenate([onesm, -2.0 * y], axis=1)

        d1 = jax.lax.dot_general(xa, yb, (((1,), (1,)), ((), ())),
                                 preferred_element_type=f32)
        d2t = jax.lax.dot_general(ya, xb, (((1,), (1,)), ((), ())),
                                  preferred_element_type=f32)

        m1 = jnp.min(d1, axis=0, keepdims=True)
        idx1 = jnp.min(jnp.where(d1 <= m1, iif, f32(ne)), axis=0,
                       keepdims=True)
        h1 = (iif == idx1).astype(f32)
        aligned1 = jax.lax.dot_general(h1, x, (((0,), (0,)), ((), ())),
                                       preferred_element_type=f32)
        acc1 = acc1 + jnp.abs(aligned1 - y)

        m2t = jnp.min(d2t, axis=0, keepdims=True)
        idx2 = jnp.min(jnp.where(d2t <= m2t, iif, f32(ne)), axis=0,
                       keepdims=True)
        h2t = (iif == idx2).astype(f32)
        aligned2 = jax.lax.dot_general(h2t, y, (((0,), (0,)), ((), ())),
                                       preferred_element_type=f32)
        acc2 = acc2 + jnp.abs(aligned2 - x)

    acc1_ref[:] = acc1
    acc2_ref[:] = acc2

    ax = px[:, :_ACTION_DIM]
    ay = py[:, :_ACTION_DIM]
    al = jnp.sum(jnp.abs(ax - ay), axis=1, keepdims=True) / f32(_ACTION_DIM)
    kvec = jax.lax.broadcasted_iota(jnp.int32, (bbatch, 1), 0) + g * bbatch
    is1 = jnp.mod(kvec, horizon) == 1
    w = jnp.where(is1, f32(_ACTION_WEIGHT), f32(1.0))
    asum_ref[0] += jnp.sum(al * w)
    asum_ref[1] += jnp.sum(jnp.where(is1, al, f32(0.0)))

    @pl.when(g == nsteps - 1)
    def _flush():
        csum = (_TARGET_WEIGHT * jnp.sum(acc1_ref[:]) + jnp.sum(acc2_ref[:]))
        li = jax.lax.broadcasted_iota(jnp.int32, (1, 128), 1)
        out_ref[0] = (jnp.where(li == 0, csum, f32(0.0))
                      + jnp.where(li == 1, asum_ref[0], f32(0.0))
                      + jnp.where(li == 2, asum_ref[1], f32(0.0)))


def kernel(preds, targ):
    bs, horizon, td = preds.shape
    nb = bs * horizon
    ne = (td - _ACTION_DIM) // _OBS_DIM
    od = _OBS_DIM

    p2 = preds.reshape(nb, td)
    t2 = targ.reshape(nb, td)

    bbatch = 16
    grid = (nb // bbatch,)
    body = functools.partial(_chamfer_body, horizon, bbatch, ne, od)
    out = pl.pallas_call(
        body,
        grid=grid,
        in_specs=[
            pl.BlockSpec((bbatch, td), lambda g: (g, 0)),
            pl.BlockSpec((bbatch, td), lambda g: (g, 0)),
        ],
        out_specs=pl.BlockSpec((1, 1, 128), lambda g: (0, 0, 0)),
        out_shape=jax.ShapeDtypeStruct((1, 1, 128), jnp.float32),
        scratch_shapes=[
            pltpu.VMEM((ne, od), jnp.float32),
            pltpu.VMEM((ne, od), jnp.float32),
            pltpu.SMEM((2,), jnp.float32),
        ],
    )(p2, t2)

    csum = out[0, 0, 0]
    acts = out[0, 0, 1]
    a0s = out[0, 0, 2]

    chamfer_loss = csum / (_TARGET_WEIGHT + 1.0) / (nb * ne * od)
    action_loss = acts / nb
    a0_loss = a0s / bs
    loss = action_loss + chamfer_loss
    return (loss, a0_loss)


# speedup vs baseline: 1.0560x; 1.0560x over previous
import functools

import jax
import jax.numpy as jnp
from jax.experimental import pallas as pl
from jax.experimental.pallas import tpu as pltpu

_ACTION_WEIGHT = 10.0
_ACTION_DIM = 8
_OBS_DIM = 128
_TARGET_WEIGHT = 3.0


def _entities_block(pm, bbatch, ne, od):
    ad = _ACTION_DIM
    f32 = jnp.float32
    c = pm[:, :ne * od].reshape(bbatch * ne, od)
    main = c[:, ad:]
    wrapsrc = jnp.concatenate(
        [c[1:, :ad], jnp.zeros((1, ad), f32)], axis=0)
    tails = pm[:, ne * od:ne * od + ad]
    trep = jnp.broadcast_to(tails[:, None, :],
                            (bbatch, ne, ad)).reshape(bbatch * ne, ad)
    rows = jax.lax.broadcasted_iota(jnp.int32, (bbatch * ne, 1), 0)
    islast = (rows % ne) == (ne - 1)
    wrap = jnp.where(islast, trep, wrapsrc)
    return jnp.concatenate([main, wrap], axis=1)


def _chamfer_body(horizon, bbatch, ne, od, px_ref, py_ref,
                  out_ref, acc1_ref, acc2_ref, asum_ref):
    g = pl.program_id(0)
    nsteps = pl.num_programs(0)
    f32 = jnp.float32

    iif = jax.lax.broadcasted_iota(jnp.int32, (ne, ne), 0).astype(f32)
    jjf = jax.lax.broadcasted_iota(jnp.int32, (ne, ne), 1).astype(f32)
    onesm = jnp.ones((ne, od), f32)

    @pl.when(g == 0)
    def _init():
        acc1_ref[:] = jnp.zeros_like(acc1_ref)
        acc2_ref[:] = jnp.zeros_like(acc2_ref)
        asum_ref[0] = f32(0.0)
        asum_ref[1] = f32(0.0)

    px = px_ref[:]
    py = py_ref[:]
    ex = _entities_block(px, bbatch, ne, od)
    ey = _entities_block(py, bbatch, ne, od)

    acc1 = acc1_ref[:]
    acc2 = acc2_ref[:]
    for r in range(bbatch):
        x = ex[r * ne:(r + 1) * ne]
        y = ey[r * ne:(r + 1) * ne]
        xa = jnp.concatenate([x * x, x], axis=1)
        ya = jnp.concatenate([y * y, y], axis=1)
        xb = jnp.concatenate([onesm, -2.0 * x], axis=1)
        yb = jnp.concatenate([onesm, -2.0 * y], axis=1)

        d1 = jax.lax.dot_general(xa, yb, (((1,), (1,)), ((), ())),
                                 preferred_element_type=f32)
        d2t = jax.lax.dot_general(ya, xb, (((1,), (1,)), ((), ())),
                                  preferred_element_type=f32)

        m1 = jnp.min(d1, axis=0, keepdims=True)
        idx1 = jnp.min(jnp.where(d1 <= m1, iif, f32(ne)), axis=0,
                       keepdims=True)
        h1t = (jjf == idx1.reshape(ne, 1)).astype(f32)
        aligned1 = jax.lax.dot_general(h1t, x, (((1,), (0,)), ((), ())),
                                       preferred_element_type=f32)
        acc1 = acc1 + jnp.abs(aligned1 - y)

        m2t = jnp.min(d2t, axis=0, keepdims=True)
        idx2 = jnp.min(jnp.where(d2t <= m2t, iif, f32(ne)), axis=0,
                       keepdims=True)
        h2 = (jjf == idx2.reshape(ne, 1)).astype(f32)
        aligned2 = jax.lax.dot_general(h2, y, (((1,), (0,)), ((), ())),
                                       preferred_element_type=f32)
        acc2 = acc2 + jnp.abs(aligned2 - x)

    acc1_ref[:] = acc1
    acc2_ref[:] = acc2

    ax = px[:, :_ACTION_DIM]
    ay = py[:, :_ACTION_DIM]
    al = jnp.sum(jnp.abs(ax - ay), axis=1, keepdims=True) / f32(_ACTION_DIM)
    kvec = jax.lax.broadcasted_iota(jnp.int32, (bbatch, 1), 0) + g * bbatch
    is1 = jnp.mod(kvec, horizon) == 1
    w = jnp.where(is1, f32(_ACTION_WEIGHT), f32(1.0))
    asum_ref[0] += jnp.sum(al * w)
    asum_ref[1] += jnp.sum(jnp.where(is1, al, f32(0.0)))

    @pl.when(g == nsteps - 1)
    def _flush():
        csum = (_TARGET_WEIGHT * jnp.sum(acc1_ref[:]) + jnp.sum(acc2_ref[:]))
        li = jax.lax.broadcasted_iota(jnp.int32, (1, 128), 1)
        out_ref[0] = (jnp.where(li == 0, csum, f32(0.0))
                      + jnp.where(li == 1, asum_ref[0], f32(0.0))
                      + jnp.where(li == 2, asum_ref[1], f32(0.0)))


def kernel(preds, targ):
    bs, horizon, td = preds.shape
    nb = bs * horizon
    ne = (td - _ACTION_DIM) // _OBS_DIM
    od = _OBS_DIM

    p2 = preds.reshape(nb, td)
    t2 = targ.reshape(nb, td)

    bbatch = 16
    grid = (nb // bbatch,)
    body = functools.partial(_chamfer_body, horizon, bbatch, ne, od)
    out = pl.pallas_call(
        body,
        grid=grid,
        in_specs=[
            pl.BlockSpec((bbatch, td), lambda g: (g, 0)),
            pl.BlockSpec((bbatch, td), lambda g: (g, 0)),
        ],
        out_specs=pl.BlockSpec((1, 1, 128), lambda g: (0, 0, 0)),
        out_shape=jax.ShapeDtypeStruct((1, 1, 128), jnp.float32),
        scratch_shapes=[
            pltpu.VMEM((ne, od), jnp.float32),
            pltpu.VMEM((ne, od), jnp.float32),
            pltpu.SMEM((2,), jnp.float32),
        ],
    )(p2, t2)

    csum = out[0, 0, 0]
    acts = out[0, 0, 1]
    a0s = out[0, 0, 2]

    chamfer_loss = csum / (_TARGET_WEIGHT + 1.0) / (nb * ne * od)
    action_loss = acts / nb
    a0_loss = a0s / bs
    loss = action_loss + chamfer_loss
    return (loss, a0_loss)


# speedup vs baseline: 1.0820x; 1.0246x over previous
import functools

import jax
import jax.numpy as jnp
from jax.experimental import pallas as pl
from jax.experimental.pallas import tpu as pltpu

_ACTION_WEIGHT = 10.0
_ACTION_DIM = 8
_OBS_DIM = 128
_TARGET_WEIGHT = 3.0


def _entities_block(pm, bbatch, ne, od):
    ad = _ACTION_DIM
    f32 = jnp.float32
    c = pm[:, :ne * od].reshape(bbatch * ne, od)
    main = c[:, ad:]
    wrapsrc = jnp.concatenate(
        [c[1:, :ad], jnp.zeros((1, ad), f32)], axis=0)
    tails = pm[:, ne * od:ne * od + ad]
    trep = jnp.broadcast_to(tails[:, None, :],
                            (bbatch, ne, ad)).reshape(bbatch * ne, ad)
    rows = jax.lax.broadcasted_iota(jnp.int32, (bbatch * ne, 1), 0)
    islast = (rows % ne) == (ne - 1)
    wrap = jnp.where(islast, trep, wrapsrc)
    return jnp.concatenate([main, wrap], axis=1)


def _chamfer_body(horizon, bbatch, ne, od, px_ref, py_ref,
                  out_ref, acc1_ref, acc2_ref, asum_ref):
    g = pl.program_id(0)
    nsteps = pl.num_programs(0)
    f32 = jnp.float32

    iif = jax.lax.broadcasted_iota(jnp.int32, (ne, ne), 0).astype(f32)
    jjf = jax.lax.broadcasted_iota(jnp.int32, (ne, ne), 1).astype(f32)
    onesm = jnp.ones((ne, od), f32)

    @pl.when(g == 0)
    def _init():
        acc1_ref[:] = jnp.zeros_like(acc1_ref)
        acc2_ref[:] = jnp.zeros_like(acc2_ref)
        asum_ref[0] = f32(0.0)
        asum_ref[1] = f32(0.0)

    px = px_ref[:]
    py = py_ref[:]
    ex = _entities_block(px, bbatch, ne, od)
    ey = _entities_block(py, bbatch, ne, od)

    acc1 = acc1_ref[:]
    acc2 = acc2_ref[:]
    for r in range(bbatch):
        x = ex[r * ne:(r + 1) * ne]
        y = ey[r * ne:(r + 1) * ne]
        xa = jnp.concatenate([x * x, x], axis=1)
        ya = jnp.concatenate([y * y, y], axis=1)
        xb = jnp.concatenate([onesm, -2.0 * x], axis=1)
        yb = jnp.concatenate([onesm, -2.0 * y], axis=1)

        d1 = jax.lax.dot_general(xa, yb, (((1,), (1,)), ((), ())),
                                 preferred_element_type=f32)
        d2t = jax.lax.dot_general(ya, xb, (((1,), (1,)), ((), ())),
                                  preferred_element_type=f32)

        m1 = jnp.min(d1, axis=0, keepdims=True)
        idx1 = jnp.min(jnp.where(d1 <= m1, iif, f32(ne)), axis=0,
                       keepdims=True)
        h1t = (jjf == idx1.reshape(ne, 1)).astype(f32)
        aligned1 = jax.lax.dot_general(h1t, x, (((1,), (0,)), ((), ())),
                                       preferred_element_type=f32)
        acc1 = acc1 + jnp.abs(aligned1 - y)

        m2t = jnp.min(d2t, axis=0, keepdims=True)
        idx2 = jnp.min(jnp.where(d2t <= m2t, iif, f32(ne)), axis=0,
                       keepdims=True)
        h2 = (jjf == idx2.reshape(ne, 1)).astype(f32)
        aligned2 = jax.lax.dot_general(h2, y, (((1,), (0,)), ((), ())),
                                       preferred_element_type=f32)
        acc2 = acc2 + jnp.abs(aligned2 - x)

    acc1_ref[:] = acc1
    acc2_ref[:] = acc2

    ax = px[:, :_ACTION_DIM]
    ay = py[:, :_ACTION_DIM]
    al = jnp.sum(jnp.abs(ax - ay), axis=1, keepdims=True) / f32(_ACTION_DIM)
    kvec = jax.lax.broadcasted_iota(jnp.int32, (bbatch, 1), 0) + g * bbatch
    is1 = jnp.mod(kvec, horizon) == 1
    w = jnp.where(is1, f32(_ACTION_WEIGHT), f32(1.0))
    asum_ref[0] += jnp.sum(al * w)
    asum_ref[1] += jnp.sum(jnp.where(is1, al, f32(0.0)))

    @pl.when(g == nsteps - 1)
    def _flush():
        csum = (_TARGET_WEIGHT * jnp.sum(acc1_ref[:]) + jnp.sum(acc2_ref[:]))
        li = jax.lax.broadcasted_iota(jnp.int32, (1, 128), 1)
        out_ref[0] = (jnp.where(li == 0, csum, f32(0.0))
                      + jnp.where(li == 1, asum_ref[0], f32(0.0))
                      + jnp.where(li == 2, asum_ref[1], f32(0.0)))


def kernel(preds, targ):
    bs, horizon, td = preds.shape
    nb = bs * horizon
    ne = (td - _ACTION_DIM) // _OBS_DIM
    od = _OBS_DIM

    p2 = preds.reshape(nb, td)
    t2 = targ.reshape(nb, td)

    bbatch = 32
    grid = (nb // bbatch,)
    body = functools.partial(_chamfer_body, horizon, bbatch, ne, od)
    out = pl.pallas_call(
        body,
        grid=grid,
        in_specs=[
            pl.BlockSpec((bbatch, td), lambda g: (g, 0)),
            pl.BlockSpec((bbatch, td), lambda g: (g, 0)),
        ],
        out_specs=pl.BlockSpec((1, 1, 128), lambda g: (0, 0, 0)),
        out_shape=jax.ShapeDtypeStruct((1, 1, 128), jnp.float32),
        scratch_shapes=[
            pltpu.VMEM((ne, od), jnp.float32),
            pltpu.VMEM((ne, od), jnp.float32),
            pltpu.SMEM((2,), jnp.float32),
        ],
    )(p2, t2)

    csum = out[0, 0, 0]
    acts = out[0, 0, 1]
    a0s = out[0, 0, 2]

    chamfer_loss = csum / (_TARGET_WEIGHT + 1.0) / (nb * ne * od)
    action_loss = acts / nb
    a0_loss = a0s / bs
    loss = action_loss + chamfer_loss
    return (loss, a0_loss)
